# Initial kernel scaffold; baseline (speedup 1.0000x reference)
import jax
import jax.numpy as jnp
from jax.experimental import pallas as pl


def kernel(probs, labels):
    raise NotImplementedError("write your pallas kernel here")


# speedup vs baseline: 44.0697x; 44.0697x over previous
import functools

import jax
import jax.numpy as jnp
from jax import lax
from jax.experimental import pallas as pl
from jax.experimental.pallas import tpu as pltpu
from jax.experimental.pallas import tpu_sc as plsc

_EXP_LO = 18
_OFFSET = (127 - _EXP_LO) << 23
_SHIFT = 15
_OFF2 = _OFFSET - (1 << _SHIFT)
_IDX_MAX = ((0x3F800000 - _OFFSET) >> _SHIFT) + 1
_G = 4736
_ROWS = 20
_HW = _ROWS * _G
_CHUNK = _HW // 16

_C = 19
_B = 4
_PLANE = 512 * 512
_TSLICE = _PLANE // 16
_HALF = _TSLICE // 2

_MESH = plsc.VectorSubcoreMesh(core_axis_name="c", subcore_axis_name="s")


@functools.partial(
    pl.kernel,
    out_type=jax.ShapeDtypeStruct((32 * _HW,), jnp.float32),
    mesh=_MESH,
    compiler_params=pltpu.CompilerParams(needs_layout_passes=False),
    scratch_types=[
        pltpu.VMEM((_HW,), jnp.float32),
        pltpu.VMEM((_TSLICE,), jnp.int32),
        pltpu.VMEM((_HALF,), jnp.float32),
    ],
)
def _sc_hist(probs_hbm, labels_hbm, out_hbm, hist, lab_v, pbuf):
    k = lax.axis_index("c")
    s = lax.axis_index("s")

    zeros16 = jnp.zeros((16,), jnp.float32)
    ones16 = jnp.full((16,), 1.0, jnp.float32)

    def zbody(i, carry):
        hist[pl.ds(i * 16, 16)] = zeros16
        return carry

    lax.fori_loop(0, _HW // 16, zbody, 0)

    for b in range(_B):
        loff = pl.multiple_of(b * _PLANE + s * _TSLICE, _TSLICE)
        pltpu.sync_copy(labels_hbm.at[pl.ds(loff, _TSLICE)], lab_v)
        for cl in range(10):
            c = k * 10 + cl
            rowb0 = jnp.int32(2 * cl * _G)
            rowb1 = jnp.int32((2 * cl + 1) * _G)

            def do_class(c=c, rowb0=rowb0, rowb1=rowb1):
                for half in range(2):
                    poff = pl.multiple_of(
                        (b * _C + c) * _PLANE + s * _TSLICE + half * _HALF,
                        _HALF)
                    pltpu.sync_copy(probs_hbm.at[pl.ds(poff, _HALF)], pbuf)
                    lbase = half * _HALF

                    def body(i, carry):
                        lab = lab_v[pl.ds(lbase + i * 16, 16)]
                        p = pbuf[pl.ds(i * 16, 16)]
                        fg = lab == c
                        e = jnp.where(fg, 1.0 - p, p)
                        bits = lax.bitcast_convert_type(e, jnp.int32)
                        idx = jnp.minimum(
                            jnp.maximum((bits - _OFF2) >> _SHIFT, 0),
                            _IDX_MAX)
                        rowb = jnp.where(fg, rowb1, rowb0)
                        plsc.addupdate_scatter(hist, [rowb + idx], ones16)
                        return carry

                    lax.fori_loop(0, _HALF // 16, body, 0)

            if cl == 9:
                @pl.when(k == 0)
                def _():
                    do_class()
            else:
                do_class()

    wid = k * 16 + s
    ooff = pl.multiple_of(wid * _HW, _HW)
    pltpu.sync_copy(hist, out_hbm.at[pl.ds(ooff, _HW)])


def _cumsum_last(x, nblk):
    r = x.shape[0]
    xb = x.reshape(r, nblk, 128)
    ii = lax.broadcasted_iota(jnp.int32, (128, 128), 0)
    jj = lax.broadcasted_iota(jnp.int32, (128, 128), 1)
    tri = (ii <= jj).astype(jnp.float32)
    pin = lax.dot_general(xb, tri, (((2,), (0,)), ((), ())),
                          precision=lax.Precision.HIGHEST)
    bs = pin[:, :, 127]
    i2 = lax.broadcasted_iota(jnp.int32, (nblk, nblk), 0)
    j2 = lax.broadcasted_iota(jnp.int32, (nblk, nblk), 1)
    tri2 = (i2 < j2).astype(jnp.float32)
    boff = lax.dot_general(bs, tri2, (((1,), (0,)), ((), ())),
                           precision=lax.Precision.HIGHEST)
    return (pin + boff[:, :, None]).reshape(r, nblk * 128)


def _bucket_value(gi):
    raw = gi - 1
    mb = _OFFSET + (raw << _SHIFT) + (1 << (_SHIFT - 1))
    v = lax.bitcast_convert_type(mb, jnp.float32)
    v = jnp.minimum(v, 1.0)
    v = jnp.where(gi == 0, jnp.float32(2.0 ** -19), v)
    return jnp.where(gi < 0, jnp.float32(0.0), v)


def _tc_scan_body(hist_ref, out_ref):
    x = hist_ref[...]
    y = x.reshape(20, 2, _G)
    cnt_b = y[:, 0, :]
    cnt_a = y[:, 1, :]
    m = jnp.sum(cnt_a, axis=1, keepdims=True)
    cnt_n = cnt_a + cnt_b
    tot = jnp.sum(cnt_n, axis=1, keepdims=True)
    pa = _cumsum_last(cnt_a, _G // 128)
    pn = _cumsum_last(cnt_n, _G // 128)
    sa = m - pa + cnt_a
    sn = tot - pn + cnt_n
    jac = 1.0 - (m - sa) / jnp.maximum(m + sn - sa, 1.0)
    j00 = 1.0 - m / jnp.maximum(m, 1.0)
    g = lax.broadcasted_iota(jnp.int32, (1, _G), 1)
    v = _bucket_value(g)
    dv = v - _bucket_value(g - 1)
    vlast = v[0, _G - 1]
    loss = jnp.sum(dv * jac, axis=1, keepdims=True) - vlast * j00
    pres = (m > 0).astype(jnp.float32)
    num = jnp.sum(loss * pres)
    den = jnp.maximum(jnp.sum(pres), 1.0)
    out_ref[...] = jnp.reshape(num / den, (1, 1))


def _tc_reduce_body(parts_ref, out_ref):
    out_ref[...] = jnp.sum(parts_ref[...], axis=1)


def kernel(probs, labels):
    probs_flat = probs.reshape(-1)
    labels_flat = labels.reshape(-1)
    parts = _sc_hist(probs_flat, labels_flat)
    parts = parts.reshape(2, 16, _ROWS, _G)
    hist = pl.pallas_call(
        _tc_reduce_body,
        grid=(_G // 128,),
        in_specs=[pl.BlockSpec((2, 16, _ROWS, 128),
                               lambda i: (0, 0, 0, i))],
        out_specs=pl.BlockSpec((2, _ROWS, 128), lambda i: (0, 0, i)),
        out_shape=jax.ShapeDtypeStruct((2, _ROWS, _G), jnp.float32),
    )(parts)
    out = pl.pallas_call(
        _tc_scan_body,
        out_shape=jax.ShapeDtypeStruct((1, 1), jnp.float32),
    )(hist)
    return out[0, 0]
